# baseline (speedup 1.0000x reference)
import functools

import jax
import jax.numpy as jnp
from jax import lax
from jax.experimental import pallas as pl
from jax.experimental.pallas import tpu as pltpu
from jax.experimental.pallas import tpu_sc as plsc

N_NODES = 100000
HIDDEN = 128
NVOCAB = 100
BLOCK = 200
NW = 32
NBUF = 4


def _gather_sc(graph, emb):
    n = graph.shape[0]
    nblock = n // BLOCK
    nb_hi = -(-nblock // NW)
    n_hi = nblock - NW * (nb_hi - 1)
    assert n % BLOCK == 0 and nb_hi % NBUF == 0 and nb_hi >= NBUF

    info = plsc.get_sparse_core_info()
    mesh = plsc.VectorSubcoreMesh(core_axis_name="c", subcore_axis_name="s")

    @functools.partial(
        pl.kernel,
        mesh=mesh,
        out_type=jax.ShapeDtypeStruct((n, HIDDEN), jnp.float32),
        scratch_types=[
            pltpu.VMEM((nb_hi * BLOCK,), jnp.int32),
            pltpu.VMEM((NBUF, BLOCK, HIDDEN), jnp.float32),
            pltpu.VMEM_SHARED((NVOCAB, HIDDEN), jnp.float32),
        ] + [pltpu.SemaphoreType.DMA] * (2 * NBUF),
    )
    def k(emb_hbm, idx_hbm, out_hbm, idx_v, rows_v, table_sh, *sems):
        gsems, ssems = sems[:NBUF], sems[NBUF:]
        sid = lax.axis_index("s")
        wid = sid * info.num_cores + lax.axis_index("c")
        start = nb_hi * wid - jnp.maximum(wid - n_hi, 0)
        nb = jnp.where(wid < n_hi, nb_hi, nb_hi - 1)

        @pl.when(sid == 0)
        def _():
            pltpu.sync_copy(emb_hbm.at[0], table_sh)

        @pl.when(wid < n_hi)
        def _():
            pltpu.sync_copy(
                idx_hbm.at[pl.ds(pl.multiple_of(start * BLOCK, 8), nb_hi * BLOCK)],
                idx_v.at[pl.ds(0, nb_hi * BLOCK)])

        @pl.when(wid >= n_hi)
        def _():
            pltpu.sync_copy(
                idx_hbm.at[pl.ds(pl.multiple_of(start * BLOCK, 8), (nb_hi - 1) * BLOCK)],
                idx_v.at[pl.ds(0, (nb_hi - 1) * BLOCK)])
        plsc.subcore_barrier()

        def gather(b, j):
            return pltpu.make_async_copy(
                table_sh.at[idx_v.at[pl.ds(pl.multiple_of(b * BLOCK, 8), BLOCK)]],
                rows_v.at[j],
                gsems[j])

        def store(b, j):
            return pltpu.make_async_copy(
                rows_v.at[j],
                out_hbm.at[pl.ds(pl.multiple_of((start + b) * BLOCK, 8), BLOCK)],
                ssems[j])

        def owned(b):
            return b < nb

        def gather_start(b, j):
            pl.when(owned(b))(lambda: gather(b, j).start())

        gather_start(0, 0)
        gather_start(1, 1)

        def body(o, _):
            for j in range(NBUF):
                b = o * NBUF + j
                jn = (j + 2) % NBUF

                @pl.when(owned(b))
                def _():
                    @pl.when(b >= 2)
                    def _():
                        store(b - 2, jn).wait()
                    gather_start(b + 2, jn)
                    gather(b, j).wait()
                    store(b, j).start()
            return ()

        lax.fori_loop(0, nb_hi // NBUF, body, ())

        @pl.when(wid < n_hi)
        def _():
            store(nb_hi - 2, (nb_hi - 2) % NBUF).wait()
            store(nb_hi - 1, (nb_hi - 1) % NBUF).wait()

        @pl.when(wid >= n_hi)
        def _():
            store(nb_hi - 3, (nb_hi - 3) % NBUF).wait()
            store(nb_hi - 2, (nb_hi - 2) % NBUF).wait()

    return k(emb, graph)


def _sharded(graph, emb):
    ndev = jax.device_count()
    while ndev > 1 and (N_NODES % (ndev * BLOCK) != 0
                        or (-(-(N_NODES // (ndev * BLOCK)) // NW)) % NBUF != 0):
        ndev //= 2
    if ndev <= 1:
        return jax.jit(_gather_sc)(graph, emb)
    mesh = jax.make_mesh((ndev,), ("x",))
    fn = jax.shard_map(_gather_sc, mesh=mesh,
                       in_specs=(jax.P("x"), jax.P()),
                       out_specs=jax.P("x"), check_vma=False)

    @jax.jit
    def run(g, e):
        g = jax.reshard(g, jax.NamedSharding(mesh, jax.P("x")))
        e = jax.reshard(e, jax.NamedSharding(mesh, jax.P()))
        return fn(g, e)

    return run(graph, emb)


def kernel(graph, emb):
    return _sharded(graph.astype(jnp.int32), emb)


# speedup vs baseline: 10.7614x; 10.7614x over previous
import functools

import jax
import jax.numpy as jnp
from jax import lax
from jax.experimental import pallas as pl
from jax.experimental.pallas import tpu as pltpu
from jax.experimental.pallas import tpu_sc as plsc

N_NODES = 100000
HIDDEN = 128
NVOCAB = 100
BLOCK = 200
NW = 32
NBUF = 4


def _gather_sc(graph, emb):
    n = graph.shape[0]
    nblock = n // BLOCK
    nb_hi = -(-nblock // NW)
    n_hi = nblock - NW * (nb_hi - 1)
    assert n % BLOCK == 0 and nb_hi % NBUF == 0 and nb_hi >= NBUF

    info = plsc.get_sparse_core_info()
    mesh = plsc.VectorSubcoreMesh(core_axis_name="c", subcore_axis_name="s")

    @functools.partial(
        pl.kernel,
        mesh=mesh,
        out_type=jax.ShapeDtypeStruct((n, HIDDEN), jnp.float32),
        scratch_types=[
            pltpu.VMEM((nb_hi * BLOCK,), jnp.int32),
            pltpu.VMEM((NBUF, BLOCK, HIDDEN), jnp.float32),
            pltpu.VMEM_SHARED((NVOCAB, HIDDEN), jnp.float32),
        ] + [pltpu.SemaphoreType.DMA] * (2 * NBUF),
    )
    def k(emb_hbm, idx_hbm, out_hbm, idx_v, rows_v, table_sh, *sems):
        gsems, ssems = sems[:NBUF], sems[NBUF:]
        sid = lax.axis_index("s")
        wid = sid * info.num_cores + lax.axis_index("c")
        start = nb_hi * wid - jnp.maximum(wid - n_hi, 0)
        nb = jnp.where(wid < n_hi, nb_hi, nb_hi - 1)

        @pl.when(sid == 0)
        def _():
            pltpu.sync_copy(emb_hbm.at[0], table_sh)

        @pl.when(wid < n_hi)
        def _():
            pltpu.sync_copy(
                idx_hbm.at[pl.ds(pl.multiple_of(start * BLOCK, 8), nb_hi * BLOCK)],
                idx_v.at[pl.ds(0, nb_hi * BLOCK)])

        @pl.when(wid >= n_hi)
        def _():
            pltpu.sync_copy(
                idx_hbm.at[pl.ds(pl.multiple_of(start * BLOCK, 8), (nb_hi - 1) * BLOCK)],
                idx_v.at[pl.ds(0, (nb_hi - 1) * BLOCK)])
        plsc.subcore_barrier()

        def gather(b, j):
            return pltpu.make_async_copy(
                table_sh.at[idx_v.at[pl.ds(pl.multiple_of(b * BLOCK, 8), BLOCK)]],
                rows_v.at[j],
                gsems[j])

        def store(b, j):
            return pltpu.make_async_copy(
                rows_v.at[j],
                out_hbm.at[pl.ds(pl.multiple_of((start + b) * BLOCK, 8), BLOCK)],
                ssems[j])

        def owned(b):
            return b < nb

        def gather_start(b, j):
            pl.when(owned(b))(lambda: gather(b, j).start())

        gather_start(0, 0)
        gather_start(1, 1)

        def body(o, _):
            for j in range(NBUF):
                b = o * NBUF + j
                jn = (j + 2) % NBUF

                @pl.when(owned(b))
                def _():
                    @pl.when(b >= 2)
                    def _():
                        store(b - 2, jn).wait()
                    gather_start(b + 2, jn)
                    gather(b, j).wait()
                    store(b, j).start()
            return ()

        lax.fori_loop(0, nb_hi // NBUF, body, ())

        @pl.when(wid < n_hi)
        def _():
            store(nb_hi - 2, (nb_hi - 2) % NBUF).wait()
            store(nb_hi - 1, (nb_hi - 1) % NBUF).wait()

        @pl.when(wid >= n_hi)
        def _():
            store(nb_hi - 3, (nb_hi - 3) % NBUF).wait()
            store(nb_hi - 2, (nb_hi - 2) % NBUF).wait()

    return k(emb, graph)


def kernel(graph, emb):
    return jax.jit(_gather_sc)(graph.astype(jnp.int32), emb)
